# baseline (speedup 1.0000x reference)
import functools

import jax
import jax.numpy as jnp
from jax import lax
from jax.experimental import pallas as pl
from jax.experimental.pallas import tpu as pltpu
from jax.experimental.pallas import tpu_sc as plsc

B, T, P, E, V = 4096, 50, 8, 64, 1000
N = B * T
NC, NS = 2, 16
NW = NC * NS
TOK_W = N // NW
CHUNK = 400
NCH = TOK_W // CHUNK
L = 16
WPR = E // 2


def _tree_sum(vals):
    while len(vals) > 1:
        vals = [vals[i] + vals[i + 1] for i in range(0, len(vals) - 1, 2)] + (
            [vals[-1]] if len(vals) % 2 else [])
    return vals[0]


def _body(ids_hbm, tbl_hbm, out_hbm, rcp_hbm, tbl_v, ids0, ids1, out0, out1,
          rc0, rc1, is0, is1, os0, os1, rs0, rs1):
    wid = lax.axis_index("s") * NC + lax.axis_index("c")
    ids_bufs = [ids0, ids1]
    out_bufs = [out0, out1]
    rcp_bufs = [rc0, rc1]
    isems = [is0, is1]
    osems = [os0, os1]
    rsems = [rs0, rs1]

    iota = lax.iota(jnp.int32, L)
    offs = [g * L + iota for g in range(2)]
    splat_idx = [jnp.full((L, 1), k, jnp.int32) for k in range(L)]
    gdn = lax.GatherDimensionNumbers(offset_dims=(),
                                     collapsed_slice_dims=(0,),
                                     start_index_map=(0,))

    def splat(vec, k):
        return lax.gather(vec, splat_idx[k], gdn, (1,),
                          mode=lax.GatherScatterMode.PROMISE_IN_BOUNDS)

    ids_base = wid * (TOK_W * P)
    out_base = wid * (TOK_W * E)
    rcp_base = wid * TOK_W
    lane01 = iota < 2

    def start_ids(c):
        return pltpu.async_copy(
            ids_hbm.at[pl.ds(ids_base + c * (CHUNK * P), CHUNK * P)],
            ids_bufs[c % 2], isems[c % 2])

    h_ids = start_ids(0)
    pltpu.sync_copy(tbl_hbm, tbl_v)

    h_out = [None, None]
    h_rcp = [None, None]
    for c in range(NCH):
        h_ids.wait()
        if c + 1 < NCH:
            h_ids = start_ids(c + 1)
        if h_out[c % 2] is not None:
            h_out[c % 2].wait()
            h_rcp[c % 2].wait()

        idsbuf = ids_bufs[c % 2]
        outbuf = out_bufs[c % 2]
        rcpbuf = rcp_bufs[c % 2]

        def pair_body(j, _, idsbuf=idsbuf, outbuf=outbuf, rcpbuf=rcpbuf):
            idsv = idsbuf[pl.ds(j * L, L)]
            rows = idsv * WPR
            m = (idsv != 0).astype(jnp.int32)
            cum = plsc.cumsum(m)
            c0 = splat(cum, 7)
            c1 = splat(cum, 15) - c0
            r0 = 1.0 / jnp.maximum(c0.astype(jnp.float32), 1.0)
            r1 = 1.0 / jnp.maximum(c1.astype(jnp.float32), 1.0)
            rv = jnp.where(iota == 0, r0, r1)
            plsc.store_scatter(rcpbuf, [2 * j + iota], rv, mask=lane01)
            for t in range(2):
                obase = j * (2 * E) + t * E
                sps = [splat(rows, t * 8 + p) for p in range(8)]
                for g in range(2):
                    vals = [
                        plsc.bitcast(
                            plsc.load_gather(tbl_v, [sps[p] + offs[g]]),
                            jnp.bfloat16)
                        for p in range(8)
                    ]
                    s = _tree_sum(vals)
                    a, b = plsc.unpack(s, format=plsc.PackFormat.INTERLEAVED)
                    outbuf[pl.ds(obase + g * L, L)] = a
                    outbuf[pl.ds(obase + 32 + g * L, L)] = b
            return _

        lax.fori_loop(0, CHUNK // 2, pair_body, None)

        h_out[c % 2] = pltpu.async_copy(
            outbuf,
            out_hbm.at[pl.ds(out_base + c * (CHUNK * E), CHUNK * E)],
            osems[c % 2])
        h_rcp[c % 2] = pltpu.async_copy(
            rcpbuf,
            rcp_hbm.at[pl.ds(rcp_base + c * CHUNK, CHUNK)],
            rsems[c % 2])

    for s in (0, 1):
        h_out[s].wait()
        h_rcp[s].wait()


@functools.partial(pl.kernel,
                   out_type=(jax.ShapeDtypeStruct((N * E,), jnp.float32),
                             jax.ShapeDtypeStruct((N,), jnp.float32)),
                   mesh=plsc.VectorSubcoreMesh(core_axis_name="c",
                                               subcore_axis_name="s"),
                   compiler_params=pltpu.CompilerParams(
                       needs_layout_passes=False),
                   scratch_types=[
                       pltpu.VMEM((V * WPR,), jnp.int32),
                       pltpu.VMEM((CHUNK * P,), jnp.int32),
                       pltpu.VMEM((CHUNK * P,), jnp.int32),
                       pltpu.VMEM((CHUNK * E,), jnp.float32),
                       pltpu.VMEM((CHUNK * E,), jnp.float32),
                       pltpu.VMEM((CHUNK,), jnp.float32),
                       pltpu.VMEM((CHUNK,), jnp.float32),
                       pltpu.SemaphoreType.DMA,
                       pltpu.SemaphoreType.DMA,
                       pltpu.SemaphoreType.DMA,
                       pltpu.SemaphoreType.DMA,
                       pltpu.SemaphoreType.DMA,
                       pltpu.SemaphoreType.DMA,
                   ])
def _pooled_embed(ids_hbm, tbl_hbm, out_hbm, rcp_hbm, *scratch):
    _body(ids_hbm, tbl_hbm, out_hbm, rcp_hbm, *scratch)


def kernel(phone_ids, embed_table):
    tb = embed_table.astype(jnp.bfloat16)
    packed = lax.bitcast_convert_type(
        jnp.stack([tb[:, :32], tb[:, 32:]], axis=-1), jnp.int32)
    sums, rcp = _pooled_embed(phone_ids.reshape(-1), packed.reshape(-1))
    return sums.reshape(B, T, E) * rcp.reshape(B, T, 1)


# speedup vs baseline: 1.2505x; 1.2505x over previous
import functools

import jax
import jax.numpy as jnp
from jax import lax
from jax.experimental import pallas as pl
from jax.experimental.pallas import tpu as pltpu
from jax.experimental.pallas import tpu_sc as plsc

B, T, P, E, V = 4096, 50, 8, 64, 1000
N = B * T
NC, NS = 2, 16
NW = NC * NS
TOK_W = N // NW
CHUNK = 400
CB = CHUNK // T
NCH = TOK_W // CHUNK
L = 16
WPR = E // 2


def _tree_sum(vals):
    while len(vals) > 1:
        vals = [vals[i] + vals[i + 1] for i in range(0, len(vals) - 1, 2)] + (
            [vals[-1]] if len(vals) % 2 else [])
    return vals[0]


def _body(ids_hbm, tbl_hbm, out_hbm, tbl_v, ids0, ids1, out0, out1,
          is0, is1, os0, os1):
    wid = lax.axis_index("s") * NC + lax.axis_index("c")
    ids_bufs = [ids0, ids1]
    out_bufs = [out0, out1]
    isems = [is0, is1]
    osems = [os0, os1]

    iota = lax.iota(jnp.int32, L)
    offs = [g * L + iota for g in range(2)]
    splat_idx = [jnp.full((L, 1), k, jnp.int32) for k in range(L)]
    gdn = lax.GatherDimensionNumbers(offset_dims=(),
                                     collapsed_slice_dims=(0,),
                                     start_index_map=(0,))

    def splat(vec, k):
        return lax.gather(vec, splat_idx[k], gdn, (1,),
                          mode=lax.GatherScatterMode.PROMISE_IN_BOUNDS)

    ids_base = wid * (TOK_W * P)
    out_row = wid * (TOK_W // T)

    def start_ids(c):
        return pltpu.async_copy(
            ids_hbm.at[pl.ds(ids_base + c * (CHUNK * P), CHUNK * P)],
            ids_bufs[c % 2], isems[c % 2])

    h_ids = start_ids(0)
    pltpu.sync_copy(tbl_hbm, tbl_v)

    h_out = [None, None]
    for c in range(NCH):
        h_ids.wait()
        if c + 1 < NCH:
            h_ids = start_ids(c + 1)
        if h_out[c % 2] is not None:
            h_out[c % 2].wait()

        idsbuf = ids_bufs[c % 2]
        outbuf = out_bufs[c % 2]

        def pair_body(j, _, idsbuf=idsbuf, outbuf=outbuf):
            idsv = idsbuf[pl.ds(j * L, L)]
            rows = idsv * WPR
            m = (idsv != 0).astype(jnp.int32)
            cum = plsc.cumsum(m)
            c0 = splat(cum, 7)
            c1 = splat(cum, 15) - c0
            r0 = 1.0 / jnp.maximum(c0.astype(jnp.float32), 1.0)
            r1 = 1.0 / jnp.maximum(c1.astype(jnp.float32), 1.0)
            br = j // (T // 2)
            brv = jnp.full((L,), br, jnp.int32)
            for t in range(2):
                rr = r0 if t == 0 else r1
                tok = (j % (T // 2)) * 2 + t
                tokv = jnp.full((L,), tok, jnp.int32)
                sps = [splat(rows, t * 8 + p) for p in range(8)]
                for g in range(2):
                    vals = [
                        plsc.bitcast(
                            plsc.load_gather(tbl_v, [sps[p] + offs[g]]),
                            jnp.bfloat16)
                        for p in range(8)
                    ]
                    s = _tree_sum(vals)
                    a, b = plsc.unpack(s, format=plsc.PackFormat.INTERLEAVED)
                    plsc.store_scatter(outbuf, [brv, tokv, offs[g]], a * rr)
                    plsc.store_scatter(outbuf, [brv, tokv, 32 + offs[g]],
                                       b * rr)
            return _

        lax.fori_loop(0, CHUNK // 2, pair_body, None)

        h_out[c % 2] = pltpu.async_copy(
            outbuf,
            out_hbm.at[pl.ds(out_row + c * CB, CB)],
            osems[c % 2])

    h_out[(NCH - 2) % 2].wait()
    h_out[(NCH - 1) % 2].wait()


@functools.partial(pl.kernel,
                   out_type=jax.ShapeDtypeStruct((B, T, E), jnp.float32),
                   mesh=plsc.VectorSubcoreMesh(core_axis_name="c",
                                               subcore_axis_name="s"),
                   compiler_params=pltpu.CompilerParams(
                       needs_layout_passes=False,
                       use_tc_tiling_on_sc=False),
                   scratch_types=[
                       pltpu.VMEM((V * WPR,), jnp.int32),
                       pltpu.VMEM((CHUNK * P,), jnp.int32),
                       pltpu.VMEM((CHUNK * P,), jnp.int32),
                       pltpu.VMEM((CB, T, E), jnp.float32),
                       pltpu.VMEM((CB, T, E), jnp.float32),
                       pltpu.SemaphoreType.DMA,
                       pltpu.SemaphoreType.DMA,
                       pltpu.SemaphoreType.DMA,
                       pltpu.SemaphoreType.DMA,
                   ])
def _pooled_embed(ids_hbm, tbl_hbm, out_hbm, *scratch):
    _body(ids_hbm, tbl_hbm, out_hbm, *scratch)


def kernel(phone_ids, embed_table):
    tb = embed_table.astype(jnp.bfloat16)
    packed = lax.bitcast_convert_type(
        jnp.stack([tb[:, :32], tb[:, 32:]], axis=-1), jnp.int32)
    return _pooled_embed(phone_ids.reshape(-1), packed.reshape(-1))
